# baseline (device time: 141705 ns/iter reference)
import jax
import jax.numpy as jnp
from jax import lax
from jax.experimental import pallas as pl
from jax.experimental.pallas import tpu as pltpu

N_DEV = 4
M_BLK = 1024
K_BLK = 1024
K = 4096
N = 8192
NT = N // 1024
NSLOT = 2
NW = 36


def kernel(x, w_mat, scale_x, scale_w):
    def body(x_hbm, w_hbm, sx_ref, sw_ref, out_ref,
             stage_ref, xq_ref, xa_ref, wq_ref,
             stage_sems, send_sems, recv_sems):
        me = lax.axis_index("i")
        right = lax.rem(me + 1, N_DEV)
        opp = lax.rem(me + 2, N_DEV)
        left = lax.rem(me + 3, N_DEV)

        barrier_sem = pltpu.get_barrier_semaphore()
        for nbr in (right, opp, left):
            pl.semaphore_signal(barrier_sem, inc=1, device_id=(nbr,),
                                device_id_type=pl.DeviceIdType.MESH)
        pl.semaphore_wait(barrier_sem, 3)

        dst_slot = xa_ref.at[:, pl.ds(me * K_BLK, K_BLK)]
        send_dsts = [right, left, opp]
        x_order = [right, left, opp, me]

        descs = []
        for i, blk in enumerate(x_order):
            descs.append(pltpu.make_async_copy(
                x_hbm.at[pl.ds(blk * M_BLK, M_BLK), :],
                stage_ref.at[i % NSLOT],
                stage_sems.at[i % NSLOT]))
        for j in range(32):
            t, r = j // 4, j % 4
            g = 4 + j
            descs.append(pltpu.make_async_copy(
                w_hbm.at[pl.ds(r * K_BLK, K_BLK), pl.ds(t * 1024, 1024)],
                stage_ref.at[g % NSLOT],
                stage_sems.at[g % NSLOT]))

        def wait_recv(k):
            pltpu.make_async_remote_copy(
                src_ref=xq_ref.at[k],
                dst_ref=dst_slot,
                send_sem=send_sems.at[k],
                recv_sem=recv_sems.at[k],
                device_id=(me,),
                device_id_type=pl.DeviceIdType.MESH,
            ).wait_recv()

        sends = []
        for g in range(NSLOT):
            descs[g].start()
        for g in range(NW):
            with jax.named_scope("stage_wait"):
                descs[g].wait()
            if g < 4:
                with jax.named_scope("xcast"):
                    if g < 3:
                        xq_ref[g] = (
                            stage_ref[g % NSLOT].astype(jnp.float8_e4m3fn))
                    else:
                        xa_ref[:, pl.ds(me * K_BLK, K_BLK)] = (
                            stage_ref[g % NSLOT].astype(jnp.float8_e4m3fn))
                if g + NSLOT < NW:
                    descs[g + NSLOT].start()
                if g < 3:
                    rdma = pltpu.make_async_remote_copy(
                        src_ref=xq_ref.at[g],
                        dst_ref=dst_slot,
                        send_sem=send_sems.at[g],
                        recv_sem=recv_sems.at[g],
                        device_id=(send_dsts[g],),
                        device_id_type=pl.DeviceIdType.MESH,
                    )
                    rdma.start()
                    sends.append(rdma)
            else:
                j = g - 4
                t, r = j // 4, j % 4
                with jax.named_scope("wcast"):
                    wq_ref[pl.ds(r * K_BLK, K_BLK), :] = (
                        stage_ref[g % NSLOT].astype(jnp.float8_e4m3fn))
                if g + NSLOT < NW:
                    descs[g + NSLOT].start()
                if r == 3:
                    if t == 0:
                        with jax.named_scope("recv_all"):
                            wait_recv(0)
                            wait_recv(1)
                            wait_recv(2)
                    with jax.named_scope("dot"):
                        y = lax.dot_general(
                            xa_ref[:, :], wq_ref[:, :],
                            (((1,), (0,)), ((), ())),
                            preferred_element_type=jnp.float32)
                        y = y * (sx_ref[0] * sw_ref[0])
                        out_ref[:, pl.ds(t * 1024, 1024)] = (
                            y * (1.0 / (1.0 + jnp.exp(-y))))

        for rdma in sends:
            rdma.wait_send()

    return pl.pallas_call(
        body,
        out_shape=jax.ShapeDtypeStruct((M_BLK, N), jnp.float32),
        in_specs=[
            pl.BlockSpec(memory_space=pl.ANY),
            pl.BlockSpec(memory_space=pl.ANY),
            pl.BlockSpec(memory_space=pltpu.SMEM),
            pl.BlockSpec(memory_space=pltpu.SMEM),
        ],
        out_specs=pl.BlockSpec(memory_space=pltpu.VMEM),
        scratch_shapes=[
            pltpu.VMEM((NSLOT, 1024, 1024), jnp.float32),
            pltpu.VMEM((3, M_BLK, K_BLK), jnp.float8_e4m3fn),
            pltpu.VMEM((M_BLK, K), jnp.float8_e4m3fn),
            pltpu.VMEM((K, 1024), jnp.float8_e4m3fn),
            pltpu.SemaphoreType.DMA((NSLOT,)),
            pltpu.SemaphoreType.DMA((3,)),
            pltpu.SemaphoreType.DMA((3,)),
        ],
        compiler_params=pltpu.CompilerParams(
            collective_id=0, vmem_limit_bytes=64 * 1024 * 1024),
    )(x, w_mat, scale_x, scale_w)


# device time: 100175 ns/iter; 1.4146x vs baseline; 1.4146x over previous
import jax
import jax.numpy as jnp
from jax import lax
from jax.experimental import pallas as pl
from jax.experimental.pallas import tpu as pltpu

N_DEV = 4
M_BLK = 1024
K_BLK = 1024
N = 8192
NT = N // 1024
NSLOT = 3


def kernel(x, w_mat, scale_x, scale_w):
    def body(x_hbm, w_hbm, sx_ref, sw_ref, out_ref,
             stage_ref, xq_ref, xg_ref, wq_ref,
             stage_sems, send_sems, recv_sems):
        me = lax.axis_index("i")
        right = lax.rem(me + 1, N_DEV)
        opp = lax.rem(me + 2, N_DEV)
        left = lax.rem(me + 3, N_DEV)

        barrier_sem = pltpu.get_barrier_semaphore()
        for nbr in (right, opp, left):
            pl.semaphore_signal(barrier_sem, inc=1, device_id=(nbr,),
                                device_id_type=pl.DeviceIdType.MESH)
        pl.semaphore_wait(barrier_sem, 3)

        x_order = [right, left, opp, me]
        k_order = [me, left, right, opp]

        descs = []
        for i, blk in enumerate(x_order):
            descs.append(pltpu.make_async_copy(
                x_hbm.at[pl.ds(blk * M_BLK, M_BLK), :],
                stage_ref.at[i % NSLOT],
                stage_sems.at[i % NSLOT]))
        for j in range(4 * NT):
            h, t = j // NT, j % NT
            g = 4 + j
            descs.append(pltpu.make_async_copy(
                w_hbm.at[pl.ds(k_order[h] * K_BLK, K_BLK),
                         pl.ds(t * 1024, 1024)],
                stage_ref.at[g % NSLOT],
                stage_sems.at[g % NSLOT]))

        def block_dot(xb, wb):
            return lax.dot_general(
                xb, wb, (((1,), (0,)), ((), ())),
                preferred_element_type=jnp.float32)

        sends = []
        for g in range(NSLOT):
            descs[g].start()
        for g in range(36):
            descs[g].wait()
            if g < 4:
                blk = x_order[g]
                xq_ref[pl.ds(blk * M_BLK, M_BLK), :] = (
                    stage_ref[g % NSLOT].astype(jnp.float8_e4m3fn))
                if g + NSLOT < 36:
                    descs[g + NSLOT].start()
                if g < 3:
                    rdma = pltpu.make_async_remote_copy(
                        src_ref=xq_ref.at[pl.ds(blk * M_BLK, M_BLK), :],
                        dst_ref=xg_ref.at[me],
                        send_sem=send_sems.at[g],
                        recv_sem=recv_sems.at[me],
                        device_id=(blk,),
                        device_id_type=pl.DeviceIdType.MESH,
                    )
                    rdma.start()
                    sends.append(rdma)
            else:
                j = g - 4
                h, t = j // NT, j % NT
                wq_ref[j % 2] = stage_ref[g % NSLOT].astype(jnp.float8_e5m2)
                if g + NSLOT < 36:
                    descs[g + NSLOT].start()
                if h == 0:
                    xb = xq_ref[pl.ds(me * M_BLK, M_BLK), :]
                else:
                    s = k_order[h]
                    if t == 0:
                        pltpu.make_async_remote_copy(
                            src_ref=xq_ref.at[pl.ds(0, M_BLK), :],
                            dst_ref=xg_ref.at[s],
                            send_sem=send_sems.at[0],
                            recv_sem=recv_sems.at[s],
                            device_id=(me,),
                            device_id_type=pl.DeviceIdType.MESH,
                        ).wait_recv()
                    xb = xg_ref[s]
                acc = block_dot(xb, wq_ref[j % 2])
                cols = pl.ds(t * 1024, 1024)
                if h == 0:
                    out_ref[:, cols] = acc
                else:
                    out_ref[:, cols] += acc

        s = sx_ref[0] * sw_ref[0]
        for c in range(NT):
            cols = pl.ds(c * 1024, 1024)
            y = out_ref[:, cols] * s
            out_ref[:, cols] = y * (1.0 / (1.0 + jnp.exp(-y)))

        for rdma in sends:
            rdma.wait_send()

    return pl.pallas_call(
        body,
        out_shape=jax.ShapeDtypeStruct((M_BLK, N), jnp.float32),
        in_specs=[
            pl.BlockSpec(memory_space=pl.ANY),
            pl.BlockSpec(memory_space=pl.ANY),
            pl.BlockSpec(memory_space=pltpu.SMEM),
            pl.BlockSpec(memory_space=pltpu.SMEM),
        ],
        out_specs=pl.BlockSpec(memory_space=pltpu.VMEM),
        scratch_shapes=[
            pltpu.VMEM((NSLOT, 1024, 1024), jnp.float32),
            pltpu.VMEM((N_DEV * M_BLK, K_BLK), jnp.float8_e4m3fn),
            pltpu.VMEM((N_DEV, M_BLK, K_BLK), jnp.float8_e4m3fn),
            pltpu.VMEM((2, K_BLK, 1024), jnp.float8_e5m2),
            pltpu.SemaphoreType.DMA((NSLOT,)),
            pltpu.SemaphoreType.DMA((3,)),
            pltpu.SemaphoreType.DMA((N_DEV,)),
        ],
        compiler_params=pltpu.CompilerParams(
            collective_id=0, vmem_limit_bytes=64 * 1024 * 1024),
    )(x, w_mat, scale_x, scale_w)


# device time: 96054 ns/iter; 1.4753x vs baseline; 1.0429x over previous
import jax
import jax.numpy as jnp
from jax import lax
from jax.experimental import pallas as pl
from jax.experimental.pallas import tpu as pltpu

N_DEV = 4
M_BLK = 1024
K_BLK = 1024
N = 8192
NT = N // 1024
NSLOT = 3


def kernel(x, w_mat, scale_x, scale_w):
    def body(x_hbm, w_hbm, sx_ref, sw_ref, out_ref,
             stage_ref, xq_ref, xg_ref, wq_ref,
             stage_sems, send_sems, recv_sems):
        me = lax.axis_index("i")
        right = lax.rem(me + 1, N_DEV)
        opp = lax.rem(me + 2, N_DEV)
        left = lax.rem(me + 3, N_DEV)

        barrier_sem = pltpu.get_barrier_semaphore()
        for nbr in (right, opp, left):
            pl.semaphore_signal(barrier_sem, inc=1, device_id=(nbr,),
                                device_id_type=pl.DeviceIdType.MESH)
        pl.semaphore_wait(barrier_sem, 3)

        x_order = [right, left, opp, me]
        k_order = [me, left, right, opp]

        descs = []
        for i, blk in enumerate(x_order):
            descs.append(pltpu.make_async_copy(
                x_hbm.at[pl.ds(blk * M_BLK, M_BLK), :],
                stage_ref.at[i % NSLOT],
                stage_sems.at[i % NSLOT]))
        for j in range(4 * NT):
            h, t = j // NT, j % NT
            g = 4 + j
            descs.append(pltpu.make_async_copy(
                w_hbm.at[pl.ds(k_order[h] * K_BLK, K_BLK),
                         pl.ds(t * 1024, 1024)],
                stage_ref.at[g % NSLOT],
                stage_sems.at[g % NSLOT]))

        def block_dot(xb, wb):
            return lax.dot_general(
                xb, wb, (((1,), (0,)), ((), ())),
                preferred_element_type=jnp.float32)

        sends = []
        for g in range(NSLOT):
            descs[g].start()
        for g in range(36):
            descs[g].wait()
            if g < 4:
                blk = x_order[g]
                xq_ref[pl.ds(blk * M_BLK, M_BLK), :] = (
                    stage_ref[g % NSLOT].astype(jnp.float8_e4m3fn))
                if g + NSLOT < 36:
                    descs[g + NSLOT].start()
                if g < 3:
                    rdma = pltpu.make_async_remote_copy(
                        src_ref=xq_ref.at[pl.ds(blk * M_BLK, M_BLK), :],
                        dst_ref=xg_ref.at[me],
                        send_sem=send_sems.at[g],
                        recv_sem=recv_sems.at[me],
                        device_id=(blk,),
                        device_id_type=pl.DeviceIdType.MESH,
                    )
                    rdma.start()
                    sends.append(rdma)
            else:
                j = g - 4
                h, t = j // NT, j % NT
                wq_ref[j % 2] = stage_ref[g % NSLOT].astype(jnp.float8_e5m2)
                if g + NSLOT < 36:
                    descs[g + NSLOT].start()
                if h == 0:
                    xb = xq_ref[pl.ds(me * M_BLK, M_BLK), :]
                else:
                    s = k_order[h]
                    if t == 0:
                        pltpu.make_async_remote_copy(
                            src_ref=xq_ref.at[pl.ds(0, M_BLK), :],
                            dst_ref=xg_ref.at[s],
                            send_sem=send_sems.at[0],
                            recv_sem=recv_sems.at[s],
                            device_id=(me,),
                            device_id_type=pl.DeviceIdType.MESH,
                        ).wait_recv()
                    xb = xg_ref[s]
                acc = block_dot(xb, wq_ref[j % 2])
                cols = pl.ds(t * 1024, 1024)
                if h == 0:
                    out_ref[:, cols] = acc
                elif h < 3:
                    out_ref[:, cols] += acc
                else:
                    y = (out_ref[:, cols] + acc) * (sx_ref[0] * sw_ref[0])
                    out_ref[:, cols] = y * (1.0 / (1.0 + jnp.exp(-y)))

        for rdma in sends:
            rdma.wait_send()

    return pl.pallas_call(
        body,
        out_shape=jax.ShapeDtypeStruct((M_BLK, N), jnp.float32),
        in_specs=[
            pl.BlockSpec(memory_space=pl.ANY),
            pl.BlockSpec(memory_space=pl.ANY),
            pl.BlockSpec(memory_space=pltpu.SMEM),
            pl.BlockSpec(memory_space=pltpu.SMEM),
        ],
        out_specs=pl.BlockSpec(memory_space=pltpu.VMEM),
        scratch_shapes=[
            pltpu.VMEM((NSLOT, 1024, 1024), jnp.float32),
            pltpu.VMEM((N_DEV * M_BLK, K_BLK), jnp.float8_e4m3fn),
            pltpu.VMEM((N_DEV, M_BLK, K_BLK), jnp.float8_e4m3fn),
            pltpu.VMEM((2, K_BLK, 1024), jnp.float8_e5m2),
            pltpu.SemaphoreType.DMA((NSLOT,)),
            pltpu.SemaphoreType.DMA((3,)),
            pltpu.SemaphoreType.DMA((N_DEV,)),
        ],
        compiler_params=pltpu.CompilerParams(
            collective_id=0, vmem_limit_bytes=64 * 1024 * 1024),
    )(x, w_mat, scale_x, scale_w)
